# initial kernel scaffold (unmeasured)
import jax
import jax.numpy as jnp
from jax import lax
from jax.experimental import pallas as pl
from jax.experimental.pallas import tpu as pltpu


def kernel(
    x,
):
    def body(*refs):
        pass

    out_shape = jax.ShapeDtypeStruct(..., jnp.float32)
    return pl.pallas_call(body, out_shape=out_shape)(...)


# baseline (device time: 19047 ns/iter reference)
import functools

import jax
import jax.numpy as jnp
from jax import lax
from jax.experimental import pallas as pl
from jax.experimental.pallas import tpu as pltpu


def kernel(x):
    m, n = x.shape

    def body(x_ref, out_ref, send_buf, recv_buf, send_sem, recv_sem):
        my_x = lax.axis_index("x")
        my_y = lax.axis_index("y")
        my_z = lax.axis_index("z")
        partner = (1 - my_x, my_y, my_z)

        barrier_sem = pltpu.get_barrier_semaphore()
        pl.semaphore_signal(
            barrier_sem, inc=1,
            device_id=partner, device_id_type=pl.DeviceIdType.MESH,
        )
        pl.semaphore_wait(barrier_sem, 1)

        send_buf[...] = x_ref[...].astype(jnp.bfloat16)

        rdma = pltpu.make_async_remote_copy(
            src_ref=send_buf,
            dst_ref=recv_buf,
            send_sem=send_sem,
            recv_sem=recv_sem,
            device_id=partner,
            device_id_type=pl.DeviceIdType.MESH,
        )
        rdma.start()
        rdma.wait()

        out_ref[...] = (
            x_ref[...] + recv_buf[...].astype(jnp.float32)
        ).astype(jnp.bfloat16)

        @functools.partial(pl.run_scoped, sem=pltpu.SemaphoreType.REGULAR)
        def _(sem):
            pl.semaphore_signal(
                sem, inc=1,
                device_id=partner, device_id_type=pl.DeviceIdType.MESH,
            )
            pl.semaphore_wait(sem, 1)

    return pl.pallas_call(
        body,
        out_shape=jax.ShapeDtypeStruct((m, n), jnp.bfloat16),
        in_specs=[pl.BlockSpec(memory_space=pltpu.VMEM)],
        out_specs=pl.BlockSpec(memory_space=pltpu.VMEM),
        scratch_shapes=[
            pltpu.VMEM((m, n), jnp.bfloat16),
            pltpu.VMEM((m, n), jnp.bfloat16),
            pltpu.SemaphoreType.DMA,
            pltpu.SemaphoreType.DMA,
        ],
        compiler_params=pltpu.CompilerParams(collective_id=0),
    )(x)


# device time: 16443 ns/iter; 1.1584x vs baseline; 1.1584x over previous
import functools

import jax
import jax.numpy as jnp
from jax import lax
from jax.experimental import pallas as pl
from jax.experimental.pallas import tpu as pltpu

C = 8


def kernel(x):
    m, n = x.shape
    half = m // 2
    rows = half // C

    def body(x_ref, out_ref, xsend, xrecv, xs_sems, xr_sems, ys_sems, yr_sems):
        my_x = lax.axis_index("x")
        my_y = lax.axis_index("y")
        my_z = lax.axis_index("z")
        partner_x = (1 - my_x, my_y, my_z)
        partner_y = (my_x, 1 - my_y, my_z)

        hy0 = my_y * half
        oy0 = (1 - my_y) * half

        barrier_sem = pltpu.get_barrier_semaphore()
        for nbr in (partner_x, partner_y):
            pl.semaphore_signal(
                barrier_sem, inc=1,
                device_id=nbr, device_id_type=pl.DeviceIdType.MESH,
            )
        pl.semaphore_wait(barrier_sem, 2)

        xsend[...] = x_ref[pl.ds(hy0, half), :].astype(jnp.bfloat16)

        x_rdmas = []
        for c in range(C):
            sl = pl.ds(c * rows, rows)
            rdma = pltpu.make_async_remote_copy(
                src_ref=xsend.at[sl, :],
                dst_ref=xrecv.at[sl, :],
                send_sem=xs_sems.at[c],
                recv_sem=xr_sems.at[c],
                device_id=partner_x,
                device_id_type=pl.DeviceIdType.MESH,
            )
            rdma.start()
            x_rdmas.append(rdma)

        y_sends = []
        y_recvs = []
        for c in range(C):
            x_rdmas[c].wait_recv()
            sl_out = pl.ds(hy0 + c * rows, rows)
            sl_in = pl.ds(c * rows, rows)
            s = x_ref[sl_out, :] + xrecv[sl_in, :].astype(jnp.float32)
            out_ref[sl_out, :] = s.astype(jnp.bfloat16)
            ysend = pltpu.make_async_remote_copy(
                src_ref=out_ref.at[sl_out, :],
                dst_ref=out_ref.at[sl_out, :],
                send_sem=ys_sems.at[c],
                recv_sem=yr_sems.at[c],
                device_id=partner_y,
                device_id_type=pl.DeviceIdType.MESH,
            )
            ysend.start()
            y_sends.append(ysend)
            sl_miss = pl.ds(oy0 + c * rows, rows)
            y_recvs.append(
                pltpu.make_async_remote_copy(
                    src_ref=out_ref.at[sl_miss, :],
                    dst_ref=out_ref.at[sl_miss, :],
                    send_sem=ys_sems.at[c],
                    recv_sem=yr_sems.at[c],
                    device_id=partner_y,
                    device_id_type=pl.DeviceIdType.MESH,
                )
            )

        for c in range(C):
            y_recvs[c].wait_recv()
        for c in range(C):
            x_rdmas[c].wait_send()
            y_sends[c].wait_send()

        @functools.partial(pl.run_scoped, sem=pltpu.SemaphoreType.REGULAR)
        def _(sem):
            for nbr in (partner_x, partner_y):
                pl.semaphore_signal(
                    sem, inc=1,
                    device_id=nbr, device_id_type=pl.DeviceIdType.MESH,
                )
            pl.semaphore_wait(sem, 2)

    return pl.pallas_call(
        body,
        out_shape=jax.ShapeDtypeStruct((m, n), jnp.bfloat16),
        in_specs=[pl.BlockSpec(memory_space=pltpu.VMEM)],
        out_specs=pl.BlockSpec(memory_space=pltpu.VMEM),
        scratch_shapes=[
            pltpu.VMEM((half, n), jnp.bfloat16),
            pltpu.VMEM((half, n), jnp.bfloat16),
            pltpu.SemaphoreType.DMA((C,)),
            pltpu.SemaphoreType.DMA((C,)),
            pltpu.SemaphoreType.DMA((C,)),
            pltpu.SemaphoreType.DMA((C,)),
        ],
        compiler_params=pltpu.CompilerParams(collective_id=0),
    )(x)


# device time: 15500 ns/iter; 1.2288x vs baseline; 1.0608x over previous
import functools

import jax
import jax.numpy as jnp
from jax import lax
from jax.experimental import pallas as pl
from jax.experimental.pallas import tpu as pltpu

D = 96
F = 512 - D

_FWD_SIZES = [64, 64, 64, 64, 64, 64, 32]
_OWN_SIZES = [64, 32]
_EXT_SIZES = [64, 32]
SIZES = _FWD_SIZES + _OWN_SIZES + _EXT_SIZES
CY = len(_FWD_SIZES)
CX = len(SIZES)
OFFS = [sum(SIZES[:c]) for c in range(CX)]
NSEND = sum(SIZES)


def kernel(x):
    m, n = x.shape
    half = m // 2

    def body(x_hbm, out_ref, x_vmem, xsend, xrecv, yrecv,
             in_sem, xs_sems, xr_sems, ys_sems, yr_sems):
        my_x = lax.axis_index("x")
        my_y = lax.axis_index("y")
        my_z = lax.axis_index("z")
        partner_x = (1 - my_x, my_y, my_z)
        partner_y = (my_x, 1 - my_y, my_z)

        hy0 = my_y * half
        oy0 = (1 - my_y) * half

        in_copy = pltpu.make_async_copy(x_hbm, x_vmem, in_sem)
        in_copy.start()

        barrier_sem = pltpu.get_barrier_semaphore()
        for nbr in (partner_x, partner_y):
            pl.semaphore_signal(
                barrier_sem, inc=1,
                device_id=nbr, device_id_type=pl.DeviceIdType.MESH,
            )

        def src_row(c):
            if OFFS[c] < half:
                return hy0 + OFFS[c]
            return oy0 + (OFFS[c] - half) + (half - D)

        in_copy.wait()
        pl.semaphore_wait(barrier_sem, 2)

        x_rdmas = []
        for c in range(CX):
            sl = pl.ds(OFFS[c], SIZES[c])
            xsend[sl, :] = (
                x_vmem[pl.ds(src_row(c), SIZES[c]), :].astype(jnp.bfloat16)
            )
            rdma = pltpu.make_async_remote_copy(
                src_ref=xsend.at[sl, :],
                dst_ref=xrecv.at[sl, :],
                send_sem=xs_sems.at[c],
                recv_sem=xr_sems.at[c],
                device_id=partner_x,
                device_id_type=pl.DeviceIdType.MESH,
            )
            rdma.start()
            x_rdmas.append(rdma)

        y_sends = []
        for c in range(CX):
            x_rdmas[c].wait_recv()
            sl = pl.ds(OFFS[c], SIZES[c])
            if c < CY:
                fwd = pltpu.make_async_remote_copy(
                    src_ref=xrecv.at[sl, :],
                    dst_ref=yrecv.at[sl, :],
                    send_sem=ys_sems.at[c],
                    recv_sem=yr_sems.at[c],
                    device_id=partner_y,
                    device_id_type=pl.DeviceIdType.MESH,
                )
                fwd.start()
                y_sends.append(fwd)
            rows = pl.ds(src_row(c), SIZES[c])
            s = x_vmem[rows, :] + xrecv[sl, :].astype(jnp.float32)
            out_ref[rows, :] = s.astype(jnp.bfloat16)

        for c in range(CY):
            y_sends[c].wait_recv()
            sl = pl.ds(OFFS[c], SIZES[c])
            rows = pl.ds(oy0 + OFFS[c], SIZES[c])
            s = x_vmem[rows, :] + yrecv[sl, :].astype(jnp.float32)
            out_ref[rows, :] = s.astype(jnp.bfloat16)

        for c in range(CX):
            x_rdmas[c].wait_send()
        for c in range(CY):
            y_sends[c].wait_send()

        @functools.partial(pl.run_scoped, sem=pltpu.SemaphoreType.REGULAR)
        def _(sem):
            for nbr in (partner_x, partner_y):
                pl.semaphore_signal(
                    sem, inc=1,
                    device_id=nbr, device_id_type=pl.DeviceIdType.MESH,
                )
            pl.semaphore_wait(sem, 2)

    return pl.pallas_call(
        body,
        out_shape=jax.ShapeDtypeStruct((m, n), jnp.bfloat16),
        in_specs=[pl.BlockSpec(memory_space=pl.ANY)],
        out_specs=pl.BlockSpec(memory_space=pltpu.VMEM),
        scratch_shapes=[
            pltpu.VMEM((m, n), jnp.float32),
            pltpu.VMEM((NSEND, n), jnp.bfloat16),
            pltpu.VMEM((NSEND, n), jnp.bfloat16),
            pltpu.VMEM((F, n), jnp.bfloat16),
            pltpu.SemaphoreType.DMA,
            pltpu.SemaphoreType.DMA((CX,)),
            pltpu.SemaphoreType.DMA((CX,)),
            pltpu.SemaphoreType.DMA((CY,)),
            pltpu.SemaphoreType.DMA((CY,)),
        ],
        compiler_params=pltpu.CompilerParams(collective_id=0),
    )(x)
